# baseline (speedup 1.0000x reference)
import jax
import jax.numpy as jnp
from jax.experimental import pallas as pl
from jax.experimental.pallas import tpu as pltpu

B = 4
N_NOISY = 8192
N_CLEAN = 10000
N_CLEAN_PAD = 10048
P = 64
K = 32
KC = 4
FEAT = 128
HID = 128
NBLOCKS = 4
DSM_SIGMA = 0.01

_HI = jax.lax.Precision.HIGHEST
_BIGI = 2**30
_BIGF = 1e30


def _dot(a, b):
    return jax.lax.dot(a, b, precision=_HI, preferred_element_type=jnp.float32)


def _knn1_kernel(pidx_ref, noisy_ref, noisy_t_ref, wf1_ref, bf1_ref, wf2_ref,
                 bf2_ref, wf3_ref, bf3_ref, wc_ref, b0_ref,
                 frames_ref, q_ref, cond_ref):
    noisy = noisy_ref[0]
    nt = noisy_t_ref[0]
    pidx = pidx_ref[...]

    iota_n = jax.lax.broadcasted_iota(jnp.int32, (P, N_NOISY), 1)
    onehot_q = (iota_n == pidx).astype(jnp.float32)
    q = _dot(onehot_q, noisy)
    q_ref[0] = q

    h = jnp.maximum(_dot(q, wf1_ref[...]) + bf1_ref[...], 0.0)
    h = jnp.maximum(_dot(h, wf2_ref[...]) + bf2_ref[...], 0.0)
    feat = _dot(h, wf3_ref[...]) + bf3_ref[...]
    cond_ref[0] = _dot(feat, wc_ref[...]) + b0_ref[...]

    n2 = jnp.sum(nt * nt, axis=0, keepdims=True)
    q2 = jnp.sum(q * q, axis=1, keepdims=True)
    d2 = q2 + n2 - 2.0 * _dot(q, nt)

    def round_fn(r, d2):
        m = jnp.min(d2, axis=1, keepdims=True)
        at_min = d2 == m
        idx = jnp.min(jnp.where(at_min, iota_n, _BIGI), axis=1, keepdims=True)
        onehot = iota_n == idx
        pt = _dot(onehot.astype(jnp.float32), noisy)
        frames_ref[0, pl.ds(pl.multiple_of(r * P, P), P), :] = pt
        return jnp.where(onehot, _BIGF, d2)

    jax.lax.fori_loop(0, K, round_fn, d2)


def _score_kernel(frames_ref, q_ref, cond_ref, clean_ref, clean_t_ref,
                  wx_ref, wb_ref, bb_ref, wout_ref, bout_ref, out_ref):
    b = pl.program_id(0)
    t = pl.program_id(1)

    fr = frames_ref[0]
    clean = clean_ref[0]
    ct = clean_t_ref[0]

    c2 = jnp.sum(ct * ct, axis=0, keepdims=True)
    f2 = jnp.sum(fr * fr, axis=1, keepdims=True)
    d2 = f2 + c2 - 2.0 * _dot(fr, ct)

    qt = fr.shape[0]
    iota_c = jax.lax.broadcasted_iota(jnp.int32, (qt, N_CLEAN_PAD), 1)
    def round_fn(_, carry):
        oh_acc, d2 = carry
        m = jnp.min(d2, axis=1, keepdims=True)
        at_min = d2 == m
        idx = jnp.min(jnp.where(at_min, iota_c, _BIGI), axis=1, keepdims=True)
        onehot = iota_c == idx
        return oh_acc + onehot.astype(jnp.float32), jnp.where(onehot, _BIGF, d2)

    oh_acc, _ = jax.lax.fori_loop(
        0, KC, round_fn,
        (jnp.zeros((qt, N_CLEAN_PAD), jnp.float32), d2))
    csum = _dot(oh_acc, clean)

    grad_target = csum * (1.0 / KC) - fr

    reps = qt // P
    qv = q_ref[0]
    cond = cond_ref[0]
    qfull = jnp.concatenate([qv] * reps, axis=0)
    condfull = jnp.concatenate([cond] * reps, axis=0)

    fc = fr - qfull
    hs = jnp.maximum(_dot(fc, wx_ref[...]) + condfull, 0.0)
    for i in range(NBLOCKS):
        hs = hs + jnp.maximum(_dot(hs, wb_ref[i]) + bb_ref[i], 0.0)
    grad_pred = _dot(hs, wout_ref[...]) + bout_ref[...]

    diff = grad_target - grad_pred
    partial = jnp.sum(diff * diff).reshape(1, 1, 1)

    @pl.when(t == 0)
    def _():
        out_ref[...] = jnp.zeros((1, 1, 1), jnp.float32)

    out_ref[...] += partial


def kernel(pcl_noisy, pcl_clean, pnt_idx, Wf1, bf1, Wf2, bf2, Wf3, bf3, Wx,
           Wc, b0, Wb, bb, Wout, bout):
    pidx = pnt_idx.astype(jnp.int32).reshape(P, 1)
    noisy_t = jnp.swapaxes(pcl_noisy, 1, 2)
    clean_pad = jnp.pad(pcl_clean, ((0, 0), (0, N_CLEAN_PAD - N_CLEAN), (0, 0)),
                        constant_values=1e6)
    clean_t = jnp.swapaxes(clean_pad, 1, 2)

    frames, q, cond = pl.pallas_call(
        _knn1_kernel,
        grid=(B,),
        in_specs=[
            pl.BlockSpec((P, 1), lambda b: (0, 0)),
            pl.BlockSpec((1, N_NOISY, 3), lambda b: (b, 0, 0)),
            pl.BlockSpec((1, 3, N_NOISY), lambda b: (b, 0, 0)),
            pl.BlockSpec((3, FEAT), lambda b: (0, 0)),
            pl.BlockSpec((1, FEAT), lambda b: (0, 0)),
            pl.BlockSpec((FEAT, FEAT), lambda b: (0, 0)),
            pl.BlockSpec((1, FEAT), lambda b: (0, 0)),
            pl.BlockSpec((FEAT, FEAT), lambda b: (0, 0)),
            pl.BlockSpec((1, FEAT), lambda b: (0, 0)),
            pl.BlockSpec((FEAT, HID), lambda b: (0, 0)),
            pl.BlockSpec((1, HID), lambda b: (0, 0)),
        ],
        out_specs=[
            pl.BlockSpec((1, K * P, 3), lambda b: (b, 0, 0)),
            pl.BlockSpec((1, P, 3), lambda b: (b, 0, 0)),
            pl.BlockSpec((1, P, HID), lambda b: (b, 0, 0)),
        ],
        out_shape=[
            jax.ShapeDtypeStruct((B, K * P, 3), jnp.float32),
            jax.ShapeDtypeStruct((B, P, 3), jnp.float32),
            jax.ShapeDtypeStruct((B, P, HID), jnp.float32),
        ],
        compiler_params=pltpu.CompilerParams(
            vmem_limit_bytes=100 * 1024 * 1024,
            dimension_semantics=("parallel",)),
    )(pidx, pcl_noisy, noisy_t, Wf1, bf1.reshape(1, FEAT), Wf2,
      bf2.reshape(1, FEAT), Wf3, bf3.reshape(1, FEAT), Wc, b0.reshape(1, HID))

    QT = 256
    T = (K * P) // QT
    acc = pl.pallas_call(
        _score_kernel,
        grid=(B, T),
        in_specs=[
            pl.BlockSpec((1, QT, 3), lambda b, t: (b, t, 0)),
            pl.BlockSpec((1, P, 3), lambda b, t: (b, 0, 0)),
            pl.BlockSpec((1, P, HID), lambda b, t: (b, 0, 0)),
            pl.BlockSpec((1, N_CLEAN_PAD, 3), lambda b, t: (b, 0, 0)),
            pl.BlockSpec((1, 3, N_CLEAN_PAD), lambda b, t: (b, 0, 0)),
            pl.BlockSpec((3, HID), lambda b, t: (0, 0)),
            pl.BlockSpec((NBLOCKS, HID, HID), lambda b, t: (0, 0, 0)),
            pl.BlockSpec((NBLOCKS, 1, HID), lambda b, t: (0, 0, 0)),
            pl.BlockSpec((HID, 3), lambda b, t: (0, 0)),
            pl.BlockSpec((1, 3), lambda b, t: (0, 0)),
        ],
        out_specs=pl.BlockSpec((1, 1, 1), lambda b, t: (b, 0, 0)),
        out_shape=jax.ShapeDtypeStruct((B, 1, 1), jnp.float32),
        compiler_params=pltpu.CompilerParams(
            vmem_limit_bytes=100 * 1024 * 1024,
            dimension_semantics=("parallel", "arbitrary")),
    )(frames, q, cond, clean_pad, clean_t, Wx, Wb,
      bb.reshape(NBLOCKS, 1, HID), Wout, bout.reshape(1, 3))

    scale = 0.5 * (1.0 / DSM_SIGMA) / (B * P * K)
    return jnp.sum(acc) * scale


# speedup vs baseline: 1.6817x; 1.6817x over previous
import jax
import jax.numpy as jnp
from jax.experimental import pallas as pl
from jax.experimental.pallas import tpu as pltpu

B = 4
N_NOISY = 8192
N_CLEAN = 10000
N_CLEAN_PAD = 10240
P = 64
K = 32
KC = 4
FEAT = 128
HID = 128
NBLOCKS = 4
DSM_SIGMA = 0.01

_HI = jax.lax.Precision.HIGHEST
_BIGI = 2**30
_BIGF = 1e30


def _dot(a, b):
    return jax.lax.dot(a, b, precision=_HI, preferred_element_type=jnp.float32)


def _knn1_kernel(pidx_ref, noisy_ref, noisy_t_ref, wf1_ref, bf1_ref, wf2_ref,
                 bf2_ref, wf3_ref, bf3_ref, wc_ref, b0_ref,
                 frames_ref, q_ref, cond_ref):
    noisy = noisy_ref[0]
    nt = noisy_t_ref[0]
    pidx = pidx_ref[...]

    iota_n = jax.lax.broadcasted_iota(jnp.int32, (P, N_NOISY), 1)
    onehot_q = (iota_n == pidx).astype(jnp.float32)
    q = _dot(onehot_q, noisy)
    q_ref[0] = q

    h = jnp.maximum(_dot(q, wf1_ref[...]) + bf1_ref[...], 0.0)
    h = jnp.maximum(_dot(h, wf2_ref[...]) + bf2_ref[...], 0.0)
    feat = _dot(h, wf3_ref[...]) + bf3_ref[...]
    cond_ref[0] = _dot(feat, wc_ref[...]) + b0_ref[...]

    n2 = jnp.sum(nt * nt, axis=0, keepdims=True)
    q2 = jnp.sum(q * q, axis=1, keepdims=True)
    d2 = q2 + n2 - 2.0 * _dot(q, nt)

    def round_fn(r, d2):
        m = jnp.min(d2, axis=1, keepdims=True)
        at_min = d2 == m
        idx = jnp.min(jnp.where(at_min, iota_n, _BIGI), axis=1, keepdims=True)
        onehot = iota_n == idx
        pt = _dot(onehot.astype(jnp.float32), noisy)
        frames_ref[0, pl.ds(pl.multiple_of(r * P, P), P), :] = pt
        return jnp.where(onehot, _BIGF, d2)

    jax.lax.fori_loop(0, K, round_fn, d2)


def _score_kernel(frames_ref, q_ref, cond_ref, clean_aug_ref, clean_t_ref,
                  wx_ref, wb_ref, bb_ref, wout_ref, bout_ref, out_ref):
    t = pl.program_id(1)

    fr = frames_ref[0]
    clean_aug = clean_aug_ref[0]
    ct = clean_t_ref[0]

    c2 = jnp.sum(ct * ct, axis=0, keepdims=True)
    f2 = jnp.sum(fr * fr, axis=1, keepdims=True)
    d2 = f2 + c2 - 2.0 * _dot(fr, ct)

    qt = fr.shape[0]
    big = jnp.full((qt, 128), _BIGF, jnp.float32)
    t0, t1, t2, t3 = big, big, big, big
    for j in range(N_CLEAN_PAD // 128):
        x = d2[:, j * 128:(j + 1) * 128]
        t0, x = jnp.minimum(t0, x), jnp.maximum(t0, x)
        t1, x = jnp.minimum(t1, x), jnp.maximum(t1, x)
        t2, x = jnp.minimum(t2, x), jnp.maximum(t2, x)
        t3 = jnp.minimum(t3, x)

    dv = jnp.concatenate([t0, t1, t2, t3], axis=1)
    iota_f = jax.lax.broadcasted_iota(jnp.int32, (qt, 512), 1)
    for _ in range(KC - 1):
        m = jnp.min(dv, axis=1, keepdims=True)
        idx = jnp.min(jnp.where(dv == m, iota_f, _BIGI), axis=1, keepdims=True)
        dv = jnp.where(iota_f == idx, _BIGF, dv)
    v4 = jnp.min(dv, axis=1, keepdims=True)

    sel = jnp.where(d2 <= v4, 1.0, 0.0)
    csum = _dot(sel, clean_aug)

    grad_target = csum[:, :3] / csum[:, 3:4] - fr

    reps = qt // P
    qv = q_ref[0]
    cond = cond_ref[0]
    qfull = jnp.concatenate([qv] * reps, axis=0)
    condfull = jnp.concatenate([cond] * reps, axis=0)

    fc = fr - qfull
    hs = jnp.maximum(_dot(fc, wx_ref[...]) + condfull, 0.0)
    for i in range(NBLOCKS):
        hs = hs + jnp.maximum(_dot(hs, wb_ref[i]) + bb_ref[i], 0.0)
    grad_pred = _dot(hs, wout_ref[...]) + bout_ref[...]

    diff = grad_target - grad_pred
    partial = jnp.sum(diff * diff).reshape(1, 1, 1)

    @pl.when(t == 0)
    def _():
        out_ref[...] = jnp.zeros((1, 1, 1), jnp.float32)

    out_ref[...] += partial


def kernel(pcl_noisy, pcl_clean, pnt_idx, Wf1, bf1, Wf2, bf2, Wf3, bf3, Wx,
           Wc, b0, Wb, bb, Wout, bout):
    pidx = pnt_idx.astype(jnp.int32).reshape(P, 1)
    noisy_t = jnp.swapaxes(pcl_noisy, 1, 2)
    clean_pad = jnp.pad(pcl_clean, ((0, 0), (0, N_CLEAN_PAD - N_CLEAN), (0, 0)),
                        constant_values=1e6)
    clean_t = jnp.swapaxes(clean_pad, 1, 2)
    clean_aug = jnp.concatenate(
        [clean_pad, jnp.ones((B, N_CLEAN_PAD, 1), jnp.float32)], axis=2)

    frames, q, cond = pl.pallas_call(
        _knn1_kernel,
        grid=(B,),
        in_specs=[
            pl.BlockSpec((P, 1), lambda b: (0, 0)),
            pl.BlockSpec((1, N_NOISY, 3), lambda b: (b, 0, 0)),
            pl.BlockSpec((1, 3, N_NOISY), lambda b: (b, 0, 0)),
            pl.BlockSpec((3, FEAT), lambda b: (0, 0)),
            pl.BlockSpec((1, FEAT), lambda b: (0, 0)),
            pl.BlockSpec((FEAT, FEAT), lambda b: (0, 0)),
            pl.BlockSpec((1, FEAT), lambda b: (0, 0)),
            pl.BlockSpec((FEAT, FEAT), lambda b: (0, 0)),
            pl.BlockSpec((1, FEAT), lambda b: (0, 0)),
            pl.BlockSpec((FEAT, HID), lambda b: (0, 0)),
            pl.BlockSpec((1, HID), lambda b: (0, 0)),
        ],
        out_specs=[
            pl.BlockSpec((1, K * P, 3), lambda b: (b, 0, 0)),
            pl.BlockSpec((1, P, 3), lambda b: (b, 0, 0)),
            pl.BlockSpec((1, P, HID), lambda b: (b, 0, 0)),
        ],
        out_shape=[
            jax.ShapeDtypeStruct((B, K * P, 3), jnp.float32),
            jax.ShapeDtypeStruct((B, P, 3), jnp.float32),
            jax.ShapeDtypeStruct((B, P, HID), jnp.float32),
        ],
        compiler_params=pltpu.CompilerParams(
            vmem_limit_bytes=100 * 1024 * 1024,
            dimension_semantics=("parallel",)),
    )(pidx, pcl_noisy, noisy_t, Wf1, bf1.reshape(1, FEAT), Wf2,
      bf2.reshape(1, FEAT), Wf3, bf3.reshape(1, FEAT), Wc, b0.reshape(1, HID))

    QT = 256
    T = (K * P) // QT
    acc = pl.pallas_call(
        _score_kernel,
        grid=(B, T),
        in_specs=[
            pl.BlockSpec((1, QT, 3), lambda b, t: (b, t, 0)),
            pl.BlockSpec((1, P, 3), lambda b, t: (b, 0, 0)),
            pl.BlockSpec((1, P, HID), lambda b, t: (b, 0, 0)),
            pl.BlockSpec((1, N_CLEAN_PAD, 4), lambda b, t: (b, 0, 0)),
            pl.BlockSpec((1, 3, N_CLEAN_PAD), lambda b, t: (b, 0, 0)),
            pl.BlockSpec((3, HID), lambda b, t: (0, 0)),
            pl.BlockSpec((NBLOCKS, HID, HID), lambda b, t: (0, 0, 0)),
            pl.BlockSpec((NBLOCKS, 1, HID), lambda b, t: (0, 0, 0)),
            pl.BlockSpec((HID, 3), lambda b, t: (0, 0)),
            pl.BlockSpec((1, 3), lambda b, t: (0, 0)),
        ],
        out_specs=pl.BlockSpec((1, 1, 1), lambda b, t: (b, 0, 0)),
        out_shape=jax.ShapeDtypeStruct((B, 1, 1), jnp.float32),
        compiler_params=pltpu.CompilerParams(
            vmem_limit_bytes=100 * 1024 * 1024,
            dimension_semantics=("parallel", "arbitrary")),
    )(frames, q, cond, clean_aug, clean_t, Wx, Wb,
      bb.reshape(NBLOCKS, 1, HID), Wout, bout.reshape(1, 3))

    scale = 0.5 * (1.0 / DSM_SIGMA) / (B * P * K)
    return jnp.sum(acc) * scale


# speedup vs baseline: 2.6378x; 1.5685x over previous
import jax
import jax.numpy as jnp
from jax.experimental import pallas as pl
from jax.experimental.pallas import tpu as pltpu

B = 4
N_NOISY = 8192
N_CLEAN = 10000
N_CLEAN_PAD = 10240
P = 64
K = 32
KC = 4
FEAT = 128
HID = 128
NBLOCKS = 4
DSM_SIGMA = 0.01

_HI = jax.lax.Precision.HIGHEST
_BIGI = 2**30
_BIGF = 1e30


def _dot(a, b):
    return jax.lax.dot(a, b, precision=_HI, preferred_element_type=jnp.float32)


def _dot_mlp(a, b):
    return jax.lax.dot(a, b, preferred_element_type=jnp.float32)


def _d2_vpu(pts, pts_t, p2):
    q2 = jnp.sum(pts * pts, axis=1, keepdims=True)
    qp = (pts[:, 0:1] * pts_t[0:1, :]
          + pts[:, 1:2] * pts_t[1:2, :]
          + pts[:, 2:3] * pts_t[2:3, :])
    return q2 + p2 - 2.0 * qp


def _knn1_kernel(pidx_ref, noisy_ref, noisy_t_ref, wf1_ref, bf1_ref, wf2_ref,
                 bf2_ref, wf3_ref, bf3_ref, wc_ref, b0_ref,
                 frames_ref, q_ref, cond_ref):
    noisy = noisy_ref[0]
    nt = noisy_t_ref[0]
    pidx = pidx_ref[...]

    iota_n = jax.lax.broadcasted_iota(jnp.int32, (P, N_NOISY), 1)
    onehot_q = (iota_n == pidx).astype(jnp.float32)
    q = _dot(onehot_q, noisy)
    q_ref[0] = q

    h = jnp.maximum(_dot_mlp(q, wf1_ref[...]) + bf1_ref[...], 0.0)
    h = jnp.maximum(_dot_mlp(h, wf2_ref[...]) + bf2_ref[...], 0.0)
    feat = _dot_mlp(h, wf3_ref[...]) + bf3_ref[...]
    cond_ref[0] = _dot_mlp(feat, wc_ref[...]) + b0_ref[...]

    n2 = jnp.sum(nt * nt, axis=0, keepdims=True)
    d2 = _d2_vpu(q, nt, n2)

    def round_fn(r, d2):
        m = jnp.min(d2, axis=1, keepdims=True)
        at_min = d2 == m
        idx = jnp.min(jnp.where(at_min, iota_n, _BIGI), axis=1, keepdims=True)
        onehot = iota_n == idx
        pt = _dot(onehot.astype(jnp.float32), noisy)
        frames_ref[0, pl.ds(pl.multiple_of(r * P, P), P), :] = pt
        return jnp.where(onehot, _BIGF, d2)

    jax.lax.fori_loop(0, K, round_fn, d2)


def _score_kernel(frames_ref, q_ref, cond_ref, clean_aug_ref, clean_t_ref,
                  wx_ref, wb_ref, bb_ref, wout_ref, bout_ref, out_ref):
    t = pl.program_id(1)

    fr = frames_ref[0]
    clean_aug = clean_aug_ref[0]
    ct = clean_t_ref[0]

    c2 = jnp.sum(ct * ct, axis=0, keepdims=True)
    d2 = _d2_vpu(fr, ct, c2)

    qt = fr.shape[0]
    big = jnp.full((qt, 128), _BIGF, jnp.float32)
    t0, t1, t2, t3 = big, big, big, big
    for j in range(N_CLEAN_PAD // 128):
        x = d2[:, j * 128:(j + 1) * 128]
        t0, x = jnp.minimum(t0, x), jnp.maximum(t0, x)
        t1, x = jnp.minimum(t1, x), jnp.maximum(t1, x)
        t2, x = jnp.minimum(t2, x), jnp.maximum(t2, x)
        t3 = jnp.minimum(t3, x)

    dv = jnp.concatenate([t0, t1, t2, t3], axis=1)
    iota_f = jax.lax.broadcasted_iota(jnp.int32, (qt, 512), 1)
    for _ in range(KC - 1):
        m = jnp.min(dv, axis=1, keepdims=True)
        idx = jnp.min(jnp.where(dv == m, iota_f, _BIGI), axis=1, keepdims=True)
        dv = jnp.where(iota_f == idx, _BIGF, dv)
    v4 = jnp.min(dv, axis=1, keepdims=True)

    sel = jnp.where(d2 <= v4, 1.0, 0.0)
    csum = _dot(sel, clean_aug)

    grad_target = csum[:, :3] / csum[:, 3:4] - fr

    reps = qt // P
    qv = q_ref[0]
    cond = cond_ref[0]
    qfull = jnp.concatenate([qv] * reps, axis=0)
    condfull = jnp.concatenate([cond] * reps, axis=0)

    fc = fr - qfull
    hs = jnp.maximum(_dot_mlp(fc, wx_ref[...]) + condfull, 0.0)
    for i in range(NBLOCKS):
        hs = hs + jnp.maximum(_dot_mlp(hs, wb_ref[i]) + bb_ref[i], 0.0)
    grad_pred = _dot_mlp(hs, wout_ref[...]) + bout_ref[...]

    diff = grad_target - grad_pred
    partial = jnp.sum(diff * diff).reshape(1, 1, 1)

    @pl.when(t == 0)
    def _():
        out_ref[...] = jnp.zeros((1, 1, 1), jnp.float32)

    out_ref[...] += partial


def kernel(pcl_noisy, pcl_clean, pnt_idx, Wf1, bf1, Wf2, bf2, Wf3, bf3, Wx,
           Wc, b0, Wb, bb, Wout, bout):
    pidx = pnt_idx.astype(jnp.int32).reshape(P, 1)
    noisy_t = jnp.swapaxes(pcl_noisy, 1, 2)
    clean_pad = jnp.pad(pcl_clean, ((0, 0), (0, N_CLEAN_PAD - N_CLEAN), (0, 0)),
                        constant_values=1e6)
    clean_t = jnp.swapaxes(clean_pad, 1, 2)
    clean_aug = jnp.concatenate(
        [clean_pad, jnp.ones((B, N_CLEAN_PAD, 1), jnp.float32)], axis=2)

    frames, q, cond = pl.pallas_call(
        _knn1_kernel,
        grid=(B,),
        in_specs=[
            pl.BlockSpec((P, 1), lambda b: (0, 0)),
            pl.BlockSpec((1, N_NOISY, 3), lambda b: (b, 0, 0)),
            pl.BlockSpec((1, 3, N_NOISY), lambda b: (b, 0, 0)),
            pl.BlockSpec((3, FEAT), lambda b: (0, 0)),
            pl.BlockSpec((1, FEAT), lambda b: (0, 0)),
            pl.BlockSpec((FEAT, FEAT), lambda b: (0, 0)),
            pl.BlockSpec((1, FEAT), lambda b: (0, 0)),
            pl.BlockSpec((FEAT, FEAT), lambda b: (0, 0)),
            pl.BlockSpec((1, FEAT), lambda b: (0, 0)),
            pl.BlockSpec((FEAT, HID), lambda b: (0, 0)),
            pl.BlockSpec((1, HID), lambda b: (0, 0)),
        ],
        out_specs=[
            pl.BlockSpec((1, K * P, 3), lambda b: (b, 0, 0)),
            pl.BlockSpec((1, P, 3), lambda b: (b, 0, 0)),
            pl.BlockSpec((1, P, HID), lambda b: (b, 0, 0)),
        ],
        out_shape=[
            jax.ShapeDtypeStruct((B, K * P, 3), jnp.float32),
            jax.ShapeDtypeStruct((B, P, 3), jnp.float32),
            jax.ShapeDtypeStruct((B, P, HID), jnp.float32),
        ],
        compiler_params=pltpu.CompilerParams(
            vmem_limit_bytes=100 * 1024 * 1024,
            dimension_semantics=("parallel",)),
    )(pidx, pcl_noisy, noisy_t, Wf1, bf1.reshape(1, FEAT), Wf2,
      bf2.reshape(1, FEAT), Wf3, bf3.reshape(1, FEAT), Wc, b0.reshape(1, HID))

    QT = 256
    T = (K * P) // QT
    acc = pl.pallas_call(
        _score_kernel,
        grid=(B, T),
        in_specs=[
            pl.BlockSpec((1, QT, 3), lambda b, t: (b, t, 0)),
            pl.BlockSpec((1, P, 3), lambda b, t: (b, 0, 0)),
            pl.BlockSpec((1, P, HID), lambda b, t: (b, 0, 0)),
            pl.BlockSpec((1, N_CLEAN_PAD, 4), lambda b, t: (b, 0, 0)),
            pl.BlockSpec((1, 3, N_CLEAN_PAD), lambda b, t: (b, 0, 0)),
            pl.BlockSpec((3, HID), lambda b, t: (0, 0)),
            pl.BlockSpec((NBLOCKS, HID, HID), lambda b, t: (0, 0, 0)),
            pl.BlockSpec((NBLOCKS, 1, HID), lambda b, t: (0, 0, 0)),
            pl.BlockSpec((HID, 3), lambda b, t: (0, 0)),
            pl.BlockSpec((1, 3), lambda b, t: (0, 0)),
        ],
        out_specs=pl.BlockSpec((1, 1, 1), lambda b, t: (b, 0, 0)),
        out_shape=jax.ShapeDtypeStruct((B, 1, 1), jnp.float32),
        compiler_params=pltpu.CompilerParams(
            vmem_limit_bytes=100 * 1024 * 1024,
            dimension_semantics=("parallel", "arbitrary")),
    )(frames, q, cond, clean_aug, clean_t, Wx, Wb,
      bb.reshape(NBLOCKS, 1, HID), Wout, bout.reshape(1, 3))

    scale = 0.5 * (1.0 / DSM_SIGMA) / (B * P * K)
    return jnp.sum(acc) * scale


# speedup vs baseline: 3.4304x; 1.3005x over previous
import jax
import jax.numpy as jnp
from jax.experimental import pallas as pl
from jax.experimental.pallas import tpu as pltpu

B = 4
N_NOISY = 8192
N_CLEAN = 10000
N_CLEAN_PAD = 10240
P = 64
K = 32
KC = 4
FEAT = 128
HID = 128
NBLOCKS = 4
DSM_SIGMA = 0.01

_HI = jax.lax.Precision.HIGHEST
_BIGI = 2**30
_BIGF = 1e30


def _dot(a, b):
    return jax.lax.dot(a, b, precision=_HI, preferred_element_type=jnp.float32)


def _dot_mlp(a, b):
    return jax.lax.dot(a, b, preferred_element_type=jnp.float32)


def _d2_vpu(pts, pts_t, p2):
    q2 = jnp.sum(pts * pts, axis=1, keepdims=True)
    qp = (pts[:, 0:1] * pts_t[0:1, :]
          + pts[:, 1:2] * pts_t[1:2, :]
          + pts[:, 2:3] * pts_t[2:3, :])
    return q2 + p2 - 2.0 * qp


def _knn1_kernel(pidx_ref, noisy_ref, noisy_t_ref, wf1_ref, bf1_ref, wf2_ref,
                 bf2_ref, wf3_ref, bf3_ref, wc_ref, b0_ref,
                 frames_ref, q_ref, cond_ref):
    noisy = noisy_ref[0]
    nt = noisy_t_ref[0]
    pidx = pidx_ref[...]

    iota_n = jax.lax.broadcasted_iota(jnp.int32, (P, N_NOISY), 1)
    onehot_q = (iota_n == pidx).astype(jnp.float32)
    q = _dot(onehot_q, noisy)
    q_ref[0] = q

    h = jnp.maximum(_dot_mlp(q, wf1_ref[...]) + bf1_ref[...], 0.0)
    h = jnp.maximum(_dot_mlp(h, wf2_ref[...]) + bf2_ref[...], 0.0)
    feat = _dot_mlp(h, wf3_ref[...]) + bf3_ref[...]
    cond_ref[0] = _dot_mlp(feat, wc_ref[...]) + b0_ref[...]

    n2 = jnp.sum(nt * nt, axis=0, keepdims=True)
    d2 = _d2_vpu(q, nt, n2)

    def round_fn(r, d2):
        m = jnp.min(d2, axis=1, keepdims=True)
        at_min = d2 == m
        idx = jnp.min(jnp.where(at_min, iota_n, _BIGI), axis=1, keepdims=True)
        onehot = iota_n == idx
        pt = _dot(onehot.astype(jnp.float32), noisy)
        frames_ref[0, pl.ds(pl.multiple_of(r * P, P), P), :] = pt
        return jnp.where(onehot, _BIGF, d2)

    jax.lax.fori_loop(0, K, round_fn, d2)


def _score_kernel(frames_ref, q_ref, cond_ref, clean_aug_ref, clean_t_ref,
                  wx_ref, wb_ref, bb_ref, wout_ref, bout_ref, out_ref):
    t = pl.program_id(1)

    fr = frames_ref[0]
    clean_aug = clean_aug_ref[0]
    ct = clean_t_ref[0]

    c2 = jnp.sum(ct * ct, axis=0, keepdims=True)
    d2 = _d2_vpu(fr, ct, c2)

    qt = fr.shape[0]
    big = jnp.full((qt, 128), _BIGF, jnp.float32)
    t0, t1, t2, t3 = big, big, big, big
    for j in range(N_CLEAN_PAD // 128):
        x = d2[:, j * 128:(j + 1) * 128]
        t0, x = jnp.minimum(t0, x), jnp.maximum(t0, x)
        t1, x = jnp.minimum(t1, x), jnp.maximum(t1, x)
        t2, x = jnp.minimum(t2, x), jnp.maximum(t2, x)
        t3 = jnp.minimum(t3, x)

    dv = jnp.concatenate([t0, t1, t2, t3], axis=1)
    iota_f = jax.lax.broadcasted_iota(jnp.int32, (qt, 512), 1)
    for _ in range(KC - 1):
        m = jnp.min(dv, axis=1, keepdims=True)
        idx = jnp.min(jnp.where(dv == m, iota_f, _BIGI), axis=1, keepdims=True)
        dv = jnp.where(iota_f == idx, _BIGF, dv)
    v4 = jnp.min(dv, axis=1, keepdims=True)

    sel = jnp.where(d2 <= v4, 1.0, 0.0)
    csum = _dot_mlp(sel, clean_aug)

    grad_target = csum[:, :3] / csum[:, 3:4] - fr

    reps = qt // P
    qv = q_ref[0]
    cond = cond_ref[0]
    qfull = jnp.concatenate([qv] * reps, axis=0)
    condfull = jnp.concatenate([cond] * reps, axis=0)

    fc = fr - qfull
    hs = jnp.maximum(_dot_mlp(fc, wx_ref[...]) + condfull, 0.0)
    for i in range(NBLOCKS):
        hs = hs + jnp.maximum(_dot_mlp(hs, wb_ref[i]) + bb_ref[i], 0.0)
    grad_pred = _dot_mlp(hs, wout_ref[...]) + bout_ref[...]

    diff = grad_target - grad_pred
    partial = jnp.sum(diff * diff).reshape(1, 1, 1)

    @pl.when(t == 0)
    def _():
        out_ref[...] = jnp.zeros((1, 1, 1), jnp.float32)

    out_ref[...] += partial


def kernel(pcl_noisy, pcl_clean, pnt_idx, Wf1, bf1, Wf2, bf2, Wf3, bf3, Wx,
           Wc, b0, Wb, bb, Wout, bout):
    pidx = pnt_idx.astype(jnp.int32).reshape(P, 1)
    noisy_t = jnp.swapaxes(pcl_noisy, 1, 2)
    clean_pad = jnp.pad(pcl_clean, ((0, 0), (0, N_CLEAN_PAD - N_CLEAN), (0, 0)),
                        constant_values=1e6)
    clean_t = jnp.swapaxes(clean_pad, 1, 2)
    clean_aug = jnp.concatenate(
        [clean_pad, jnp.ones((B, N_CLEAN_PAD, 1), jnp.float32)], axis=2)

    frames, q, cond = pl.pallas_call(
        _knn1_kernel,
        grid=(B,),
        in_specs=[
            pl.BlockSpec((P, 1), lambda b: (0, 0)),
            pl.BlockSpec((1, N_NOISY, 3), lambda b: (b, 0, 0)),
            pl.BlockSpec((1, 3, N_NOISY), lambda b: (b, 0, 0)),
            pl.BlockSpec((3, FEAT), lambda b: (0, 0)),
            pl.BlockSpec((1, FEAT), lambda b: (0, 0)),
            pl.BlockSpec((FEAT, FEAT), lambda b: (0, 0)),
            pl.BlockSpec((1, FEAT), lambda b: (0, 0)),
            pl.BlockSpec((FEAT, FEAT), lambda b: (0, 0)),
            pl.BlockSpec((1, FEAT), lambda b: (0, 0)),
            pl.BlockSpec((FEAT, HID), lambda b: (0, 0)),
            pl.BlockSpec((1, HID), lambda b: (0, 0)),
        ],
        out_specs=[
            pl.BlockSpec((1, K * P, 3), lambda b: (b, 0, 0)),
            pl.BlockSpec((1, P, 3), lambda b: (b, 0, 0)),
            pl.BlockSpec((1, P, HID), lambda b: (b, 0, 0)),
        ],
        out_shape=[
            jax.ShapeDtypeStruct((B, K * P, 3), jnp.float32),
            jax.ShapeDtypeStruct((B, P, 3), jnp.float32),
            jax.ShapeDtypeStruct((B, P, HID), jnp.float32),
        ],
        compiler_params=pltpu.CompilerParams(
            vmem_limit_bytes=100 * 1024 * 1024,
            dimension_semantics=("parallel",)),
    )(pidx, pcl_noisy, noisy_t, Wf1, bf1.reshape(1, FEAT), Wf2,
      bf2.reshape(1, FEAT), Wf3, bf3.reshape(1, FEAT), Wc, b0.reshape(1, HID))

    QT = 256
    T = (K * P) // QT
    acc = pl.pallas_call(
        _score_kernel,
        grid=(B, T),
        in_specs=[
            pl.BlockSpec((1, QT, 3), lambda b, t: (b, t, 0)),
            pl.BlockSpec((1, P, 3), lambda b, t: (b, 0, 0)),
            pl.BlockSpec((1, P, HID), lambda b, t: (b, 0, 0)),
            pl.BlockSpec((1, N_CLEAN_PAD, 4), lambda b, t: (b, 0, 0)),
            pl.BlockSpec((1, 3, N_CLEAN_PAD), lambda b, t: (b, 0, 0)),
            pl.BlockSpec((3, HID), lambda b, t: (0, 0)),
            pl.BlockSpec((NBLOCKS, HID, HID), lambda b, t: (0, 0, 0)),
            pl.BlockSpec((NBLOCKS, 1, HID), lambda b, t: (0, 0, 0)),
            pl.BlockSpec((HID, 3), lambda b, t: (0, 0)),
            pl.BlockSpec((1, 3), lambda b, t: (0, 0)),
        ],
        out_specs=pl.BlockSpec((1, 1, 1), lambda b, t: (b, 0, 0)),
        out_shape=jax.ShapeDtypeStruct((B, 1, 1), jnp.float32),
        compiler_params=pltpu.CompilerParams(
            vmem_limit_bytes=100 * 1024 * 1024,
            dimension_semantics=("parallel", "arbitrary")),
    )(frames, q, cond, clean_aug, clean_t, Wx, Wb,
      bb.reshape(NBLOCKS, 1, HID), Wout, bout.reshape(1, 3))

    scale = 0.5 * (1.0 / DSM_SIGMA) / (B * P * K)
    return jnp.sum(acc) * scale
